# baseline (speedup 1.0000x reference)
import functools

import jax
import jax.numpy as jnp
from jax import lax
from jax.experimental import pallas as pl
from jax.experimental.pallas import tpu as pltpu
from jax.experimental.pallas import tpu_sc as plsc

N_NODES = 10000
N_EDGES = 320000
D = 128

NC = 2
NS = 16
CHUNK = 128
IDX_BLOCK = 16
C0_BLOCKS = 5
C1_BLOCKS = 5
TOTAL_CHUNKS = NS * (C0_BLOCKS + C1_BLOCKS) * IDX_BLOCK
E_PAD = TOTAL_CHUNKS * CHUNK
N_PAD = 10240
STRIPE = N_PAD // NS

_sc_mesh = plsc.VectorSubcoreMesh(
    core_axis_name="c", subcore_axis_name="s", num_cores=NC, num_subcores=NS)


@functools.partial(
    pl.kernel,
    out_type=jax.ShapeDtypeStruct((NC, N_PAD, D), jnp.float32),
    mesh=_sc_mesh,
    scratch_types=[
        pltpu.VMEM((IDX_BLOCK, CHUNK), jnp.int32),
        pltpu.VMEM((IDX_BLOCK, CHUNK), jnp.int32),
        pltpu.VMEM((CHUNK, D), jnp.float32),
        pltpu.VMEM((CHUNK, D), jnp.float32),
        pltpu.VMEM_SHARED((N_PAD, D), jnp.float32),
        pltpu.SemaphoreType.DMA,
        pltpu.SemaphoreType.DMA,
    ],
)
def _sc_segment_sum(x_hbm, src_hbm, dst_hbm, out_hbm,
                    src_v, dst_v, rows0, rows1, accum, sg0, sg1):
    c = lax.axis_index("c")
    s = lax.axis_index("s")

    with jax.named_scope("zero"):
        def _zero_row(i, carry):
            for cc in range(D // 16):
                rows0[i, pl.ds(cc * 16, 16)] = jnp.zeros((16,), jnp.float32)
            return carry
        lax.fori_loop(0, CHUNK, _zero_row, 0)
        for t in range(STRIPE // CHUNK):
            pltpu.sync_copy(rows0, accum.at[pl.ds(s * STRIPE + t * CHUNK, CHUNK)])
        plsc.subcore_barrier()

    def _run(chunk_base, nblocks):
        for bb in range(nblocks):
            chunk0 = chunk_base + bb * IDX_BLOCK
            pltpu.sync_copy(src_hbm.at[pl.ds(chunk0, IDX_BLOCK)], src_v)
            pltpu.sync_copy(dst_hbm.at[pl.ds(chunk0, IDX_BLOCK)], dst_v)

            pltpu.async_copy(x_hbm.at[src_v.at[0]], rows0, sg0)
            pltpu.async_copy(x_hbm.at[src_v.at[1]], rows1, sg1)

            def _step(t, carry):
                j0 = 2 * t
                pltpu.make_async_copy(x_hbm.at[src_v.at[j0]], rows0, sg0).wait()
                pltpu.sync_copy(rows0, accum.at[dst_v.at[j0]], add=True)
                pltpu.async_copy(x_hbm.at[src_v.at[j0 + 2]], rows0, sg0)
                pltpu.make_async_copy(x_hbm.at[src_v.at[j0 + 1]], rows1, sg1).wait()
                pltpu.sync_copy(rows1, accum.at[dst_v.at[j0 + 1]], add=True)
                pltpu.async_copy(x_hbm.at[src_v.at[j0 + 3]], rows1, sg1)
                return carry
            lax.fori_loop(0, IDX_BLOCK // 2 - 1, _step, 0)

            last = IDX_BLOCK - 2
            pltpu.make_async_copy(x_hbm.at[src_v.at[last]], rows0, sg0).wait()
            pltpu.sync_copy(rows0, accum.at[dst_v.at[last]], add=True)
            pltpu.make_async_copy(x_hbm.at[src_v.at[last + 1]], rows1, sg1).wait()
            pltpu.sync_copy(rows1, accum.at[dst_v.at[last + 1]], add=True)

    with jax.named_scope("edges_c0"):
        @pl.when(c == 0)
        def _():
            _run(s * (C0_BLOCKS * IDX_BLOCK), C0_BLOCKS)

    with jax.named_scope("edges_c1"):
        @pl.when(c == 1)
        def _():
            _run(NS * C0_BLOCKS * IDX_BLOCK + s * (C1_BLOCKS * IDX_BLOCK),
                 C1_BLOCKS)

    with jax.named_scope("readout"):
        plsc.subcore_barrier()
        for t in range(STRIPE // CHUNK):
            off = s * STRIPE + t * CHUNK
            pltpu.sync_copy(accum.at[pl.ds(off, CHUNK)], rows0)
            pltpu.sync_copy(rows0, out_hbm.at[c].at[pl.ds(off, CHUNK)])


def _tc_body(p0_ref, p1_ref, w_ref, b_ref, o_ref):
    acc = p0_ref[0] + p1_ref[0]
    h = lax.dot_general(acc, w_ref[...], (((1,), (1,)), ((), ())),
                        preferred_element_type=jnp.float32)
    o_ref[...] = jnp.maximum(h + b_ref[...], 0.0)


_ROWS_BLK = 1000


def _tc_linear(partials, W, b2d):
    return pl.pallas_call(
        _tc_body,
        grid=(N_NODES // _ROWS_BLK,),
        in_specs=[
            pl.BlockSpec((1, _ROWS_BLK, D), lambda i: (0, i, 0)),
            pl.BlockSpec((1, _ROWS_BLK, D), lambda i: (1, i, 0)),
            pl.BlockSpec((D, D), lambda i: (0, 0)),
            pl.BlockSpec((1, D), lambda i: (0, 0)),
        ],
        out_specs=pl.BlockSpec((_ROWS_BLK, D), lambda i: (i, 0)),
        out_shape=jax.ShapeDtypeStruct((N_NODES, D), jnp.float32),
    )(partials, partials, W, b2d)


def kernel(x, edge_index, W, b):
    src = edge_index[0]
    dst = edge_index[1]
    pad = E_PAD - N_EDGES
    src_p = jnp.concatenate(
        [src, jnp.zeros((pad,), jnp.int32)]).reshape(TOTAL_CHUNKS, CHUNK)
    dst_p = jnp.concatenate(
        [dst, jnp.full((pad,), N_NODES, jnp.int32)]).reshape(TOTAL_CHUNKS, CHUNK)
    partials = _sc_segment_sum(x, src_p, dst_p)
    return _tc_linear(partials, W, b.reshape(1, D))


# speedup vs baseline: 3.6470x; 3.6470x over previous
import functools

import jax
import jax.numpy as jnp
from jax import lax
from jax.experimental import pallas as pl
from jax.experimental.pallas import tpu as pltpu
from jax.experimental.pallas import tpu_sc as plsc

N_NODES = 10000
N_EDGES = 320000
D = 128

NC = 2
NS = 16
CHUNK = 128
IDX_BLOCK = 16
C0_BLOCKS = 5
C1_BLOCKS = 5
TOTAL_CHUNKS = NS * (C0_BLOCKS + C1_BLOCKS) * IDX_BLOCK
E_PAD = TOTAL_CHUNKS * CHUNK
N_PAD = 10240
STRIPE = N_PAD // NS

_sc_mesh = plsc.VectorSubcoreMesh(
    core_axis_name="c", subcore_axis_name="s", num_cores=NC, num_subcores=NS)


@functools.partial(
    pl.kernel,
    out_type=jax.ShapeDtypeStruct((NC, N_PAD, D), jnp.float32),
    mesh=_sc_mesh,
    scratch_types=[
        pltpu.VMEM((IDX_BLOCK, CHUNK), jnp.int32),
        pltpu.VMEM((IDX_BLOCK, CHUNK), jnp.int32),
        pltpu.VMEM((CHUNK, D), jnp.float32),
        pltpu.VMEM((CHUNK, D), jnp.float32),
        pltpu.VMEM_SHARED((N_PAD, D), jnp.float32),
        pltpu.SemaphoreType.DMA,
        pltpu.SemaphoreType.DMA,
    ],
)
def _sc_segment_sum(x_hbm, src_hbm, dst_hbm, out_hbm,
                    src_v, dst_v, rows0, rows1, accum, sg0, sg1):
    c = lax.axis_index("c")
    s = lax.axis_index("s")

    with jax.named_scope("zero"):
        def _zero_row(i, carry):
            for cc in range(D // 16):
                rows0[i, pl.ds(cc * 16, 16)] = jnp.zeros((16,), jnp.float32)
            return carry
        lax.fori_loop(0, CHUNK, _zero_row, 0)
        for t in range(STRIPE // CHUNK):
            pltpu.sync_copy(rows0, accum.at[pl.ds(s * STRIPE + t * CHUNK, CHUNK)])
        plsc.subcore_barrier()

    def _run(chunk_base, nblocks):
        for bb in range(nblocks):
            chunk0 = chunk_base + bb * IDX_BLOCK
            pltpu.sync_copy(src_hbm.at[pl.ds(chunk0, IDX_BLOCK)], src_v)
            pltpu.sync_copy(dst_hbm.at[pl.ds(chunk0, IDX_BLOCK)], dst_v)

            pltpu.async_copy(x_hbm.at[src_v.at[0]], rows0, sg0)
            pltpu.async_copy(x_hbm.at[src_v.at[1]], rows1, sg1)

            def _step(t, carry):
                j0 = 2 * t
                pltpu.make_async_copy(x_hbm.at[src_v.at[j0]], rows0, sg0).wait()
                pltpu.sync_copy(rows0, accum.at[dst_v.at[j0]], add=True)
                pltpu.async_copy(x_hbm.at[src_v.at[j0 + 2]], rows0, sg0)
                pltpu.make_async_copy(x_hbm.at[src_v.at[j0 + 1]], rows1, sg1).wait()
                pltpu.sync_copy(rows1, accum.at[dst_v.at[j0 + 1]], add=True)
                pltpu.async_copy(x_hbm.at[src_v.at[j0 + 3]], rows1, sg1)
                return carry
            lax.fori_loop(0, IDX_BLOCK // 2 - 1, _step, 0)

            last = IDX_BLOCK - 2
            pltpu.make_async_copy(x_hbm.at[src_v.at[last]], rows0, sg0).wait()
            pltpu.sync_copy(rows0, accum.at[dst_v.at[last]], add=True)
            pltpu.make_async_copy(x_hbm.at[src_v.at[last + 1]], rows1, sg1).wait()
            pltpu.sync_copy(rows1, accum.at[dst_v.at[last + 1]], add=True)

    with jax.named_scope("edges_c0"):
        @pl.when(c == 0)
        def _():
            _run(s * (C0_BLOCKS * IDX_BLOCK), C0_BLOCKS)

    with jax.named_scope("edges_c1"):
        @pl.when(c == 1)
        def _():
            _run(NS * C0_BLOCKS * IDX_BLOCK + s * (C1_BLOCKS * IDX_BLOCK),
                 C1_BLOCKS)

    with jax.named_scope("readout"):
        plsc.subcore_barrier()
        for t in range(STRIPE // CHUNK):
            off = s * STRIPE + t * CHUNK
            pltpu.sync_copy(accum.at[pl.ds(off, CHUNK)], rows0)
            pltpu.sync_copy(rows0, out_hbm.at[c].at[pl.ds(off, CHUNK)])


def _tc_body(p0_ref, p1_ref, w_ref, b_ref, o_ref):
    acc = p0_ref[0] + p1_ref[0]
    h = lax.dot_general(acc, w_ref[...], (((1,), (1,)), ((), ())),
                        preferred_element_type=jnp.float32)
    o_ref[...] = jnp.maximum(h + b_ref[...], 0.0)


_ROWS_BLK = 1000


def _tc_linear(partials, W, b2d):
    return pl.pallas_call(
        _tc_body,
        grid=(N_NODES // _ROWS_BLK,),
        in_specs=[
            pl.BlockSpec((1, _ROWS_BLK, D), lambda i: (0, i, 0)),
            pl.BlockSpec((1, _ROWS_BLK, D), lambda i: (1, i, 0)),
            pl.BlockSpec((D, D), lambda i: (0, 0)),
            pl.BlockSpec((1, D), lambda i: (0, 0)),
        ],
        out_specs=pl.BlockSpec((_ROWS_BLK, D), lambda i: (i, 0)),
        out_shape=jax.ShapeDtypeStruct((N_NODES, D), jnp.float32),
    )(partials, partials, W, b2d)


def kernel(x, edge_index, W, b):
    src = edge_index[0]
    dst = edge_index[1]
    pad = E_PAD - N_EDGES
    pad_iota = jnp.arange(pad, dtype=jnp.int32)
    src_p = jnp.concatenate(
        [src, pad_iota % N_NODES]).reshape(TOTAL_CHUNKS, CHUNK)
    dst_p = jnp.concatenate(
        [dst, N_NODES + pad_iota % (N_PAD - N_NODES)]).reshape(TOTAL_CHUNKS, CHUNK)
    partials = _sc_segment_sum(x, src_p, dst_p)
    return _tc_linear(partials, W, b.reshape(1, D))


# speedup vs baseline: 3.8777x; 1.0632x over previous
import functools

import jax
import jax.numpy as jnp
from jax import lax
from jax.experimental import pallas as pl
from jax.experimental.pallas import tpu as pltpu
from jax.experimental.pallas import tpu_sc as plsc

N_NODES = 10000
N_EDGES = 320000
D = 128

NC = 2
NS = 16
CHUNK = 64
NBUF = 4
IDX_BLOCK = 32
BLOCKS_PER_TILE = 5
CHUNKS_PER_TILE = BLOCKS_PER_TILE * IDX_BLOCK
TOTAL_CHUNKS = NC * NS * CHUNKS_PER_TILE
E_PAD = TOTAL_CHUNKS * CHUNK
N_PAD = 10240
STRIPE = N_PAD // NS
ZROWS = 64

_sc_mesh = plsc.VectorSubcoreMesh(
    core_axis_name="c", subcore_axis_name="s", num_cores=NC, num_subcores=NS)


@functools.partial(
    pl.kernel,
    out_type=jax.ShapeDtypeStruct((NC, N_PAD, D), jnp.float32),
    mesh=_sc_mesh,
    scratch_types=[
        pltpu.VMEM((IDX_BLOCK, CHUNK), jnp.int32),
        pltpu.VMEM((IDX_BLOCK, CHUNK), jnp.int32),
        [pltpu.VMEM((CHUNK, D), jnp.float32)] * NBUF,
        pltpu.VMEM((ZROWS, D), jnp.float32),
        pltpu.VMEM_SHARED((N_PAD, D), jnp.float32),
        [pltpu.SemaphoreType.DMA] * NBUF,
    ],
)
def _sc_segment_sum(x_hbm, src_hbm, dst_hbm, out_hbm,
                    src_v, dst_v, rows, zbuf, accum, sems):
    c = lax.axis_index("c")
    s = lax.axis_index("s")

    with jax.named_scope("zero"):
        def _zero_row(i, carry):
            for cc in range(D // 16):
                zbuf[i, pl.ds(cc * 16, 16)] = jnp.zeros((16,), jnp.float32)
            return carry
        lax.fori_loop(0, ZROWS, _zero_row, 0)
        for t in range(STRIPE // ZROWS):
            pltpu.sync_copy(zbuf, accum.at[pl.ds(s * STRIPE + t * ZROWS, ZROWS)])
        plsc.subcore_barrier()

    with jax.named_scope("edges"):
        base = (c * NS + s) * CHUNKS_PER_TILE
        for bb in range(BLOCKS_PER_TILE):
            chunk0 = base + bb * IDX_BLOCK
            pltpu.sync_copy(src_hbm.at[pl.ds(chunk0, IDX_BLOCK)], src_v)
            pltpu.sync_copy(dst_hbm.at[pl.ds(chunk0, IDX_BLOCK)], dst_v)

            for k in range(NBUF):
                pltpu.async_copy(x_hbm.at[src_v.at[k]], rows[k], sems[k])

            def _step(t, carry):
                j0 = NBUF * t
                for k in range(NBUF):
                    j = j0 + k
                    pltpu.make_async_copy(
                        x_hbm.at[src_v.at[j]], rows[k], sems[k]).wait()
                    pltpu.sync_copy(rows[k], accum.at[dst_v.at[j]], add=True)
                    pltpu.async_copy(
                        x_hbm.at[src_v.at[j + NBUF]], rows[k], sems[k])
                return carry
            lax.fori_loop(0, IDX_BLOCK // NBUF - 1, _step, 0)

            for k in range(NBUF):
                j = IDX_BLOCK - NBUF + k
                pltpu.make_async_copy(
                    x_hbm.at[src_v.at[j]], rows[k], sems[k]).wait()
                pltpu.sync_copy(rows[k], accum.at[dst_v.at[j]], add=True)

    with jax.named_scope("readout"):
        plsc.subcore_barrier()
        for t in range(STRIPE // ZROWS):
            off = s * STRIPE + t * ZROWS
            pltpu.sync_copy(accum.at[pl.ds(off, ZROWS)], zbuf)
            pltpu.sync_copy(zbuf, out_hbm.at[c].at[pl.ds(off, ZROWS)])


def _tc_body(p0_ref, p1_ref, w_ref, b_ref, o_ref):
    acc = p0_ref[0] + p1_ref[0]
    h = lax.dot_general(acc, w_ref[...], (((1,), (1,)), ((), ())),
                        preferred_element_type=jnp.float32)
    o_ref[...] = jnp.maximum(h + b_ref[...], 0.0)


_ROWS_BLK = 1000


def _tc_linear(partials, W, b2d):
    return pl.pallas_call(
        _tc_body,
        grid=(N_NODES // _ROWS_BLK,),
        in_specs=[
            pl.BlockSpec((1, _ROWS_BLK, D), lambda i: (0, i, 0)),
            pl.BlockSpec((1, _ROWS_BLK, D), lambda i: (1, i, 0)),
            pl.BlockSpec((D, D), lambda i: (0, 0)),
            pl.BlockSpec((1, D), lambda i: (0, 0)),
        ],
        out_specs=pl.BlockSpec((_ROWS_BLK, D), lambda i: (i, 0)),
        out_shape=jax.ShapeDtypeStruct((N_NODES, D), jnp.float32),
    )(partials, partials, W, b2d)


def kernel(x, edge_index, W, b):
    src = edge_index[0]
    dst = edge_index[1]
    pad = E_PAD - N_EDGES
    pad_iota = jnp.arange(pad, dtype=jnp.int32)
    src_p = jnp.concatenate(
        [src, pad_iota % N_NODES]).reshape(TOTAL_CHUNKS, CHUNK)
    dst_p = jnp.concatenate(
        [dst, N_NODES + pad_iota % (N_PAD - N_NODES)]).reshape(TOTAL_CHUNKS, CHUNK)
    partials = _sc_segment_sum(x, src_p, dst_p)
    return _tc_linear(partials, W, b.reshape(1, D))


# speedup vs baseline: 3.9357x; 1.0150x over previous
import functools

import jax
import jax.numpy as jnp
from jax import lax
from jax.experimental import pallas as pl
from jax.experimental.pallas import tpu as pltpu
from jax.experimental.pallas import tpu_sc as plsc

N_NODES = 10000
N_EDGES = 320000
D = 128

NC = 2
NS = 16
CHUNK = 64
NBUF = 4
IDX_BLOCK = 32
TOTAL_CHUNKS = N_EDGES // CHUNK
CHUNKS_PER_TILE = 160
TAIL_CHUNKS = TOTAL_CHUNKS - 31 * CHUNKS_PER_TILE
N_PAD = 10240
STRIPE = N_PAD // NS
ZROWS = 64

_sc_mesh = plsc.VectorSubcoreMesh(
    core_axis_name="c", subcore_axis_name="s", num_cores=NC, num_subcores=NS)


@functools.partial(
    pl.kernel,
    out_type=jax.ShapeDtypeStruct((NC, N_PAD, D), jnp.float32),
    mesh=_sc_mesh,
    scratch_types=[
        pltpu.VMEM((IDX_BLOCK * CHUNK,), jnp.int32),
        pltpu.VMEM((IDX_BLOCK, CHUNK), jnp.int32),
        [pltpu.VMEM((CHUNK, D), jnp.float32)] * NBUF,
        pltpu.VMEM((ZROWS, D), jnp.float32),
        pltpu.VMEM_SHARED((N_PAD, D), jnp.float32),
        [pltpu.SemaphoreType.DMA] * NBUF,
    ],
)
def _sc_segment_sum(x_hbm, src_hbm, dst_hbm, out_hbm,
                    src_v, dst_v, rows, zbuf, accum, sems):
    c = lax.axis_index("c")
    s = lax.axis_index("s")
    w = c * NS + s

    with jax.named_scope("zero"):
        def _zero_row(i, carry):
            for cc in range(D // 16):
                zbuf[i, pl.ds(cc * 16, 16)] = jnp.zeros((16,), jnp.float32)
            return carry
        lax.fori_loop(0, ZROWS, _zero_row, 0)
        for t in range(STRIPE // ZROWS):
            pltpu.sync_copy(zbuf, accum.at[pl.ds(s * STRIPE + t * ZROWS, ZROWS)])
        plsc.subcore_barrier()

    def _run_block(chunk0, nchunks):
        pltpu.sync_copy(src_hbm.at[pl.ds(chunk0 * CHUNK, nchunks * CHUNK)],
                        src_v.at[pl.ds(0, nchunks * CHUNK)])
        pltpu.sync_copy(dst_hbm.at[pl.ds(chunk0, nchunks)],
                        dst_v.at[pl.ds(0, nchunks)])

        def _gather(j, k):
            idx = src_v.at[pl.ds(j * CHUNK, CHUNK)]
            return pltpu.make_async_copy(x_hbm.at[idx], rows[k], sems[k])

        for k in range(NBUF):
            _gather(k, k).start()

        def _step(t, carry):
            j0 = NBUF * t
            for k in range(NBUF):
                j = j0 + k
                _gather(j, k).wait()
                pltpu.sync_copy(rows[k], accum.at[dst_v.at[j]], add=True)
                _gather(j + NBUF, k).start()
            return carry
        lax.fori_loop(0, nchunks // NBUF - 1, _step, 0)

        for k in range(NBUF):
            j = nchunks - NBUF + k
            _gather(j, k).wait()
            pltpu.sync_copy(rows[k], accum.at[dst_v.at[j]], add=True)

    with jax.named_scope("edges"):
        @pl.when(w < NC * NS - 1)
        def _():
            for bb in range(CHUNKS_PER_TILE // IDX_BLOCK):
                _run_block(w * CHUNKS_PER_TILE + bb * IDX_BLOCK, IDX_BLOCK)

        @pl.when(w == NC * NS - 1)
        def _():
            base = (NC * NS - 1) * CHUNKS_PER_TILE
            _run_block(base, IDX_BLOCK)
            _run_block(base + IDX_BLOCK, TAIL_CHUNKS - IDX_BLOCK)

    with jax.named_scope("readout"):
        plsc.subcore_barrier()
        for t in range(STRIPE // ZROWS):
            off = s * STRIPE + t * ZROWS
            pltpu.sync_copy(accum.at[pl.ds(off, ZROWS)], zbuf)
            pltpu.sync_copy(zbuf, out_hbm.at[c].at[pl.ds(off, ZROWS)])


def _tc_body(p0_ref, p1_ref, w_ref, b_ref, o_ref):
    acc = p0_ref[0] + p1_ref[0]
    h = lax.dot_general(acc, w_ref[...], (((1,), (1,)), ((), ())),
                        preferred_element_type=jnp.float32)
    o_ref[...] = jnp.maximum(h + b_ref[...], 0.0)


_ROWS_BLK = 2000


def _tc_linear(partials, W, b2d):
    return pl.pallas_call(
        _tc_body,
        grid=(N_NODES // _ROWS_BLK,),
        in_specs=[
            pl.BlockSpec((1, _ROWS_BLK, D), lambda i: (0, i, 0)),
            pl.BlockSpec((1, _ROWS_BLK, D), lambda i: (1, i, 0)),
            pl.BlockSpec((D, D), lambda i: (0, 0)),
            pl.BlockSpec((1, D), lambda i: (0, 0)),
        ],
        out_specs=pl.BlockSpec((_ROWS_BLK, D), lambda i: (i, 0)),
        out_shape=jax.ShapeDtypeStruct((N_NODES, D), jnp.float32),
    )(partials, partials, W, b2d)


def kernel(x, edge_index, W, b):
    src = edge_index[0]
    dst2d = edge_index[1].reshape(TOTAL_CHUNKS, CHUNK)
    partials = _sc_segment_sum(x, src, dst2d)
    return _tc_linear(partials, W, b.reshape(1, D))


# speedup vs baseline: 4.3093x; 1.0949x over previous
import functools

import jax
import jax.numpy as jnp
from jax import lax
from jax.experimental import pallas as pl
from jax.experimental.pallas import tpu as pltpu
from jax.experimental.pallas import tpu_sc as plsc

N_NODES = 10000
N_EDGES = 320000
D = 128

NC = 2
NS = 16
CHUNK = 64
NBUF = 4
IDX_BLOCK = 32
TOTAL_CHUNKS = N_EDGES // CHUNK
CHUNKS_PER_TILE = 160
TAIL_CHUNKS = TOTAL_CHUNKS - 31 * CHUNKS_PER_TILE
N_PAD = 10240
STRIPE = N_PAD // NS
ZROWS = 64

_sc_mesh = plsc.VectorSubcoreMesh(
    core_axis_name="c", subcore_axis_name="s", num_cores=NC, num_subcores=NS)


@functools.partial(
    pl.kernel,
    out_type=jax.ShapeDtypeStruct((NC, N_PAD, D), jnp.float32),
    mesh=_sc_mesh,
    scratch_types=[
        pltpu.VMEM((IDX_BLOCK * CHUNK,), jnp.int32),
        pltpu.VMEM((IDX_BLOCK * CHUNK,), jnp.int32),
        pltpu.VMEM((IDX_BLOCK, CHUNK), jnp.int32),
        [pltpu.VMEM((CHUNK, D), jnp.float32)] * NBUF,
        pltpu.VMEM((ZROWS, D), jnp.float32),
        pltpu.VMEM_SHARED((N_PAD, D), jnp.float32),
        [pltpu.SemaphoreType.DMA] * NBUF,
    ],
)
def _sc_segment_sum(x_hbm, edge_hbm, out_hbm,
                    src_v, dst_1, dst_v, rows, zbuf, accum, sems):
    c = lax.axis_index("c")
    s = lax.axis_index("s")
    w = c * NS + s

    with jax.named_scope("zero"):
        def _zero_row(i, carry):
            for cc in range(D // 16):
                zbuf[i, pl.ds(cc * 16, 16)] = jnp.zeros((16,), jnp.float32)
            return carry
        lax.fori_loop(0, ZROWS, _zero_row, 0)
        for t in range(STRIPE // ZROWS):
            pltpu.sync_copy(zbuf, accum.at[pl.ds(s * STRIPE + t * ZROWS, ZROWS)])
        plsc.subcore_barrier()

    def _run_block(chunk0, nchunks):
        n = nchunks * CHUNK
        pltpu.sync_copy(edge_hbm.at[0].at[pl.ds(chunk0 * CHUNK, n)],
                        src_v.at[pl.ds(0, n)])
        pltpu.sync_copy(edge_hbm.at[1].at[pl.ds(chunk0 * CHUNK, n)],
                        dst_1.at[pl.ds(0, n)])
        def _rearr(i, carry):
            for q in range(CHUNK // 16):
                dst_v[i, pl.ds(q * 16, 16)] = dst_1[pl.ds(i * CHUNK + q * 16, 16)]
            return carry
        lax.fori_loop(0, nchunks, _rearr, 0)

        def _gather(j, k):
            idx = src_v.at[pl.ds(j * CHUNK, CHUNK)]
            return pltpu.make_async_copy(x_hbm.at[idx], rows[k], sems[k])

        for k in range(NBUF):
            _gather(k, k).start()

        def _step(t, carry):
            j0 = NBUF * t
            for k in range(NBUF):
                j = j0 + k
                _gather(j, k).wait()
                pltpu.sync_copy(rows[k], accum.at[dst_v.at[j]], add=True)
                _gather(j + NBUF, k).start()
            return carry
        lax.fori_loop(0, nchunks // NBUF - 1, _step, 0)

        for k in range(NBUF):
            j = nchunks - NBUF + k
            _gather(j, k).wait()
            pltpu.sync_copy(rows[k], accum.at[dst_v.at[j]], add=True)

    with jax.named_scope("edges"):
        @pl.when(w < NC * NS - 1)
        def _():
            for bb in range(CHUNKS_PER_TILE // IDX_BLOCK):
                _run_block(w * CHUNKS_PER_TILE + bb * IDX_BLOCK, IDX_BLOCK)

        @pl.when(w == NC * NS - 1)
        def _():
            base = (NC * NS - 1) * CHUNKS_PER_TILE
            _run_block(base, IDX_BLOCK)
            _run_block(base + IDX_BLOCK, TAIL_CHUNKS - IDX_BLOCK)

    with jax.named_scope("readout"):
        plsc.subcore_barrier()
        for t in range(STRIPE // ZROWS):
            off = s * STRIPE + t * ZROWS
            pltpu.sync_copy(accum.at[pl.ds(off, ZROWS)], zbuf)
            pltpu.sync_copy(zbuf, out_hbm.at[c].at[pl.ds(off, ZROWS)])


def _tc_body(p0_ref, p1_ref, w_ref, b_ref, o_ref):
    acc = p0_ref[0] + p1_ref[0]
    h = lax.dot_general(acc, w_ref[...], (((1,), (1,)), ((), ())),
                        preferred_element_type=jnp.float32)
    o_ref[...] = jnp.maximum(h + b_ref[...], 0.0)


_ROWS_BLK = 2000


def _tc_linear(partials, W, b2d):
    return pl.pallas_call(
        _tc_body,
        grid=(N_NODES // _ROWS_BLK,),
        in_specs=[
            pl.BlockSpec((1, _ROWS_BLK, D), lambda i: (0, i, 0)),
            pl.BlockSpec((1, _ROWS_BLK, D), lambda i: (1, i, 0)),
            pl.BlockSpec((D, D), lambda i: (0, 0)),
            pl.BlockSpec((1, D), lambda i: (0, 0)),
        ],
        out_specs=pl.BlockSpec((_ROWS_BLK, D), lambda i: (i, 0)),
        out_shape=jax.ShapeDtypeStruct((N_NODES, D), jnp.float32),
    )(partials, partials, W, b2d)


def kernel(x, edge_index, W, b):
    partials = _sc_segment_sum(x, edge_index)
    return _tc_linear(partials, W, b.reshape(1, D))
